# baseline (speedup 1.0000x reference)
import functools

import jax
import jax.numpy as jnp
from jax import lax
from jax.experimental import pallas as pl
from jax.experimental.pallas import tpu as pltpu
from jax.experimental.pallas import tpu_sc as plsc

NC = 2
NS = 16
K = 128
F = 128


def _mesh():
    return plsc.VectorSubcoreMesh(core_axis_name="c", subcore_axis_name="s")


def _make_deg(ch, npad):
    ntiles = NC * NS

    R = npad // 128

    def body(dst_hbm, w_hbm, outw_hbm, out1_hbm,
             dst_v, w_v, accw, acc1, acc2d, idx_v, shw, sh1):
        cid = lax.axis_index("c")
        sid = lax.axis_index("s")
        tile = cid * NS + sid
        zf = jnp.zeros((16,), jnp.float32)

        def zgrp(i, _):
            accw[pl.ds(i * 16, 16)] = zf
            acc1[pl.ds(i * 16, 16)] = zf
            return 0
        lax.fori_loop(0, npad // 16, zgrp, 0)
        for g in range(R // 16):
            idx_v[pl.ds(g * 16, 16)] = lax.iota(jnp.int32, 16) + g * 16
        @pl.when(sid < R // 8)
        def _():
            for c in range(8):
                for seg in range(128 // 16):
                    acc2d[c, pl.ds(seg * 16, 16)] = zf
            pltpu.sync_copy(acc2d.at[pl.ds(0, 8)], shw.at[pl.ds(sid * 8, 8)])
            pltpu.sync_copy(acc2d.at[pl.ds(0, 8)], sh1.at[pl.ds(sid * 8, 8)])
        pltpu.sync_copy(dst_hbm.at[pl.ds(tile * ch, ch)], dst_v)
        pltpu.sync_copy(w_hbm.at[pl.ds(tile * ch, ch)], w_v)
        ones16 = jnp.ones((16,), jnp.float32)

        def chunk(j, _):
            for e in range(K // 16):
                sl = pl.ds(e * 16, 16)
                d16 = dst_v[j, sl]
                plsc.addupdate_scatter(accw, [d16], w_v[j, sl])
                plsc.addupdate_scatter(acc1, [d16], ones16)
            return 0
        lax.fori_loop(0, ch, chunk, 0)
        plsc.subcore_barrier()

        def repack(acc_1d):
            def row(r, _):
                for seg in range(128 // 16):
                    acc2d[r, pl.ds(seg * 16, 16)] = \
                        acc_1d[pl.ds(r * 128 + seg * 16, 16)]
                return 0
            lax.fori_loop(0, R, row, 0)
        repack(accw)
        pltpu.sync_copy(acc2d, shw.at[idx_v], add=True)
        repack(acc1)
        pltpu.sync_copy(acc2d, sh1.at[idx_v], add=True)
        plsc.subcore_barrier()

        @pl.when(sid < R // 8)
        def _():
            pltpu.sync_copy(shw.at[pl.ds(sid * 8, 8)],
                            outw_hbm.at[cid, pl.ds(sid * 8, 8)])
            pltpu.sync_copy(sh1.at[pl.ds(sid * 8, 8)],
                            out1_hbm.at[cid, pl.ds(sid * 8, 8)])

    return pl.kernel(
        body,
        out_type=(jax.ShapeDtypeStruct((NC, R, 128), jnp.float32),
                  jax.ShapeDtypeStruct((NC, R, 128), jnp.float32)),
        mesh=_mesh(),
        compiler_params=pltpu.CompilerParams(needs_layout_passes=False),
        scratch_types=[
            pltpu.VMEM((ch, K), jnp.int32),
            pltpu.VMEM((ch, K), jnp.float32),
            pltpu.VMEM((npad,), jnp.float32),
            pltpu.VMEM((npad,), jnp.float32),
            pltpu.VMEM((R, 128), jnp.float32),
            pltpu.VMEM((R,), jnp.int32),
            pltpu.VMEM_SHARED((R, 128), jnp.float32),
            pltpu.VMEM_SHARED((R, 128), jnp.float32),
        ],
    )


F2 = F // 2


def _make_spmv(ch, n, npad, weighted, nt=1):
    rows_per_tile = npad // NS
    RING = 16

    def body(src_hbm, dst_hbm, *rest):
        if weighted:
            (w_hbm, x_hbm, y_hbm, srcr, dstr, wr,
             rows0, rows1, rows2, rows3,
             x_sh, accum, sem0, sem1, sem2) = rest
        else:
            (x_hbm, y_hbm, srcr, dstr,
             rows0, rows1, rows2, rows3,
             x_sh, accum, sem0, sem1, sem2) = rest
        cid = lax.axis_index("c")
        sid = lax.axis_index("s")
        zf = jnp.zeros((16,), jnp.float32)
        ebase = sid * ch
        nload = n // 10

        def stage(cbase, rbase, sync):
            srcd = pltpu.make_async_copy(
                src_hbm.at[pl.ds(ebase + cbase, 8)],
                srcr.at[pl.ds(rbase, 8)], sem2)
            srcd.start()
            dstd = pltpu.make_async_copy(
                dst_hbm.at[pl.ds(ebase + cbase, 8)],
                dstr.at[pl.ds(rbase, 8)], sem2)
            dstd.start()
            if weighted:
                wd = pltpu.make_async_copy(
                    w_hbm.at[pl.ds(ebase + cbase, 8)],
                    wr.at[pl.ds(rbase, 8)], sem2)
                wd.start()
            if sync:
                srcd.wait()
                dstd.wait()
                if weighted:
                    wd.wait()

        def wait_stage():
            pltpu.make_async_copy(src_hbm.at[pl.ds(0, 8)],
                                  srcr.at[pl.ds(0, 8)], sem2).wait()
            pltpu.make_async_copy(dst_hbm.at[pl.ds(0, 8)],
                                  dstr.at[pl.ds(0, 8)], sem2).wait()
            if weighted:
                pltpu.make_async_copy(w_hbm.at[pl.ds(0, 8)],
                                      wr.at[pl.ds(0, 8)], sem2).wait()

        def scale(rows_v, jrow):
            def grp(g, _):
                wv = wr[jrow, pl.ds(g * 16, 16)]
                for e16 in range(16):
                    ws = wv[e16]
                    row = g * 16 + e16
                    for c in range(F2 // 16):
                        sl = pl.ds(c * 16, 16)
                        rows_v[row, sl] = rows_v[row, sl] * ws
                return 0
            lax.fori_loop(0, K // 16, grp, 0)

        rows = (rows0, rows1, rows2, rows3)
        npair = ch // 8

        def halfgrp(slot4):
            for b in range(4):
                pltpu.make_async_copy(x_sh.at[srcr.at[slot4 + b]], rows[b],
                                      sem0).wait()
                if weighted:
                    scale(rows[b], slot4 + b)
                pltpu.async_copy(rows[b], accum.at[dstr.at[slot4 + b]], sem1,
                                 add=True)
            for b in range(4):
                pltpu.make_async_copy(rows[b], accum.at[dstr.at[slot4 + b]],
                                      sem1).wait()

        def firegrp(slot4):
            for b in range(4):
                pltpu.async_copy(x_sh.at[srcr.at[slot4 + b]], rows[b], sem0)

        def pairgrp(p, _):
            slot = lax.rem(p, 2) * 8
            oslot = 8 - slot

            @pl.when(p + 1 < npair)
            def _():
                wait_stage()

            halfgrp(slot)
            firegrp(slot + 4)
            halfgrp(slot + 4)

            @pl.when(p + 1 < npair)
            def _():
                firegrp(oslot)

            @pl.when(p + 2 < npair)
            def _():
                stage(p * 8 + 16, slot, False)
            return 0

        def phase(tt, _):
            def zrow(i, __):
                for c in range(F2 // 16):
                    rows0[i, pl.ds(c * 16, 16)] = zf
                return 0
            lax.fori_loop(0, K, zrow, 0)
            for b in range(rows_per_tile // K):
                pltpu.sync_copy(
                    rows0, accum.at[pl.ds(sid * rows_per_tile + b * K, K)])
            @pl.when(sid < 10)
            def _():
                pltpu.sync_copy(
                    x_hbm.at[pl.ds((tt * NC + cid) * n + sid * nload, nload)],
                    x_sh.at[pl.ds(sid * nload, nload)])
            stage(0, 0, True)
            stage(8, 8, False)
            plsc.subcore_barrier()
            firegrp(0)
            lax.fori_loop(0, npair, pairgrp, 0)
            plsc.subcore_barrier()
            pltpu.sync_copy(
                accum.at[pl.ds(sid * rows_per_tile, rows_per_tile)],
                y_hbm.at[tt * NC + cid,
                         pl.ds(sid * rows_per_tile, rows_per_tile)])
            return 0
        lax.fori_loop(0, nt, phase, 0)

    scratch = [
        pltpu.VMEM((RING, K), jnp.int32),
        pltpu.VMEM((RING, K), jnp.int32),
    ]
    if weighted:
        scratch.append(pltpu.VMEM((RING, K), jnp.float32))
    scratch += [
        pltpu.VMEM((K, F2), jnp.float32)] * 4 + [
        pltpu.VMEM_SHARED((n, F2), jnp.float32),
        pltpu.VMEM_SHARED((npad, F2), jnp.float32),
        pltpu.SemaphoreType.DMA,
        pltpu.SemaphoreType.DMA,
        pltpu.SemaphoreType.DMA,
    ]
    return pl.kernel(
        body,
        out_type=jax.ShapeDtypeStruct((nt * NC, npad, F2), jnp.float32),
        mesh=_mesh(),
        compiler_params=pltpu.CompilerParams(needs_layout_passes=False,
                                             use_tc_tiling_on_sc=False),
        scratch_types=scratch,
    )


def _prescale_body(degw_ref, deg1_ref, xs_ref,
                   xall_ref, dw_ref, d1_ref):
    dw = lax.rsqrt(degw_ref[0] + degw_ref[1] + 1.0)
    d1 = lax.rsqrt(deg1_ref[0] + deg1_ref[1] + 1.0)
    dw_ref[...] = dw
    d1_ref[...] = d1
    for t in range(xs_ref.shape[0]):
        xt = xs_ref[t] * dw
        xall_ref[t, 0] = xt[:, :F2]
        xall_ref[t, 1] = xt[:, F2:]


def _halves(y_ref, x, dinv):
    y = jnp.concatenate([y_ref[0], y_ref[1]], axis=1)
    return dinv * y + dinv * dinv * x


def _t0_body(bn, y_ref, x_ref, dw_ref, d1_ref,
             Wi2_ref, Wh2_ref, Whi_ref, Whh_ref, b2_ref, bh_ref,
             h_ref, hs_ref):
    i = pl.program_id(0)
    rows = lax.broadcasted_iota(jnp.int32, (bn, F), 0) + i * bn
    cols = lax.broadcasted_iota(jnp.int32, (bn, F), 1)
    eye = (rows == cols).astype(jnp.float32)
    P = _halves(y_ref[0], x_ref[0], dw_ref[...])
    zr = jax.nn.sigmoid(
        jnp.dot(P, Wi2_ref[...], preferred_element_type=jnp.float32)
        + jnp.dot(eye, Wh2_ref[...], preferred_element_type=jnp.float32)
        + b2_ref[...])
    z = zr[:, :F]
    r = zr[:, F:]
    rh = r * eye
    cand = jnp.tanh(
        jnp.dot(P, Whi_ref[...], preferred_element_type=jnp.float32)
        + jnp.dot(rh, Whh_ref[...], preferred_element_type=jnp.float32)
        + bh_ref[...])
    h = (1.0 - z) * rh + z * cand
    h_ref[...] = h
    hsc = d1_ref[...] * h
    hs_ref[0] = hsc[:, :F2]
    hs_ref[1] = hsc[:, F2:]


def _stepA_body(y_ref, x_ref, dw_ref, q_ref, h_ref, d1_ref,
                Wi2_ref, Wh2_ref, b2_ref,
                z_ref, rh_ref, rhs_ref):
    d1 = d1_ref[...]
    P = _halves(y_ref[0], x_ref[0], dw_ref[...])
    Q = _halves(q_ref, h_ref[...], d1)
    zr = jax.nn.sigmoid(
        jnp.dot(P, Wi2_ref[...], preferred_element_type=jnp.float32)
        + jnp.dot(Q, Wh2_ref[...], preferred_element_type=jnp.float32)
        + b2_ref[...])
    z = zr[:, :F]
    r = zr[:, F:]
    rh = r * h_ref[...]
    z_ref[...] = z
    rh_ref[...] = rh
    rhsc = d1 * rh
    rhs_ref[0] = rhsc[:, :F2]
    rhs_ref[1] = rhsc[:, F2:]


def _stepB_body(y_ref, x_ref, dw_ref, s_ref, rh_ref, z_ref, d1_ref,
                Whi_ref, Whh_ref, bh_ref, hsum_ref,
                h_ref, hs_ref, hsumo_ref):
    d1 = d1_ref[...]
    P = _halves(y_ref[0], x_ref[0], dw_ref[...])
    S = _halves(s_ref, rh_ref[...], d1)
    cand = jnp.tanh(
        jnp.dot(P, Whi_ref[...], preferred_element_type=jnp.float32)
        + jnp.dot(S, Whh_ref[...], preferred_element_type=jnp.float32)
        + bh_ref[...])
    z = z_ref[...]
    h = (1.0 - z) * rh_ref[...] + z * cand
    h_ref[...] = h
    hsc = d1 * h
    hs_ref[0] = hsc[:, :F2]
    hs_ref[1] = hsc[:, F2:]
    hsumo_ref[...] = hsum_ref[...] + h


def _pool_body(g, c, bn, hsum_ref, batch_ref, linW_ref, linb_ref,
               out_ref, sums_sc, cnt_sc):
    i = pl.program_id(0)
    oh = (batch_ref[...] == lax.broadcasted_iota(jnp.int32, (bn, g), 1)
          ).astype(jnp.float32)
    part = lax.dot_general(oh, hsum_ref[...], (((0,), (0,)), ((), ())),
                           preferred_element_type=jnp.float32)
    pcnt = lax.dot_general(oh, jnp.ones((bn, 1), jnp.float32),
                           (((0,), (0,)), ((), ())),
                           preferred_element_type=jnp.float32)

    @pl.when(i == 0)
    def _():
        sums_sc[...] = jnp.zeros_like(sums_sc)
        cnt_sc[...] = jnp.zeros_like(cnt_sc)
    sums_sc[...] += part
    cnt_sc[...] += pcnt

    @pl.when(i == pl.num_programs(0) - 1)
    def _():
        pooled = sums_sc[...] / jnp.maximum(cnt_sc[...], 1.0)
        out_ref[...] = (jnp.dot(pooled, linW_ref[...],
                                preferred_element_type=jnp.float32)
                        + linb_ref[...])


def _f32(shape):
    return jax.ShapeDtypeStruct(shape, jnp.float32)


def kernel(xs, edge_index, edge_weight, batch,
           W_zi, b_zi, W_zh, b_zh, W_ri, b_ri, W_rh, b_rh,
           W_hi, b_hi, W_hh, b_hh, lin_W, lin_b):
    T, n, f = xs.shape
    E = edge_index.shape[1]
    G = 32
    C = lin_W.shape[1]
    ntiles = NC * NS
    npad = -(-(n + 1) // (NS * K)) * (NS * K)
    ch = -(-E // (NS * K))
    ch = -(-ch // 16) * 16
    epad = NS * ch * K
    ch_deg = epad // (ntiles * K)

    src2 = jnp.pad(edge_index[0], (0, epad - E)).reshape(epad // K, K)
    dst2 = jnp.pad(edge_index[1], (0, epad - E),
                   constant_values=n).reshape(epad // K, K)
    ew2 = jnp.pad(edge_weight, (0, epad - E)).reshape(epad // K, K)

    Wi2 = jnp.concatenate([W_zi, W_ri], axis=1)
    Wh2 = jnp.concatenate([W_zh, W_rh], axis=1)
    b2 = jnp.concatenate([b_zi + b_zh, b_ri + b_rh]).reshape(1, 2 * F)
    bh2 = (b_hi + b_hh).reshape(1, F)
    batch2 = batch.reshape(n, 1)

    bn = 1000 if n % 1000 == 0 else (n // (n // 1000))
    nb = n // bn

    degw_h, deg1_h = _make_deg(ch_deg, npad)(dst2, ew2)
    degw = degw_h.reshape(NC, npad)[:, :n].reshape(NC, n, 1)
    deg1 = deg1_h.reshape(NC, npad)[:, :n].reshape(NC, n, 1)

    spmv_w = _make_spmv(ch, n, npad, True, T)
    spmv_1 = _make_spmv(ch, n, npad, False)

    def spec2(last=1):
        return pl.BlockSpec((NC, bn, last), lambda i: (0, i, 0))

    def specs():
        return pl.BlockSpec((NC, bn, F2), lambda i: (0, i, 0))

    def spec_y(t):
        return pl.BlockSpec((1, NC, bn, F2), lambda i, _t=t: (_t, 0, i, 0))

    def spec_t(t):
        return pl.BlockSpec((1, bn, F), lambda i, _t=t: (_t, i, 0))

    def specv(last=F):
        return pl.BlockSpec((bn, last), lambda i: (i, 0))

    def specw(r, c):
        return pl.BlockSpec((r, c), lambda i: (0, 0))

    xall, dw, d1 = pl.pallas_call(
        _prescale_body,
        grid=(nb,),
        in_specs=[spec2(), spec2(), pl.BlockSpec((T, bn, F), lambda i: (0, i, 0))],
        out_specs=[pl.BlockSpec((T, NC, bn, F2), lambda i: (0, 0, i, 0)),
                   specv(1), specv(1)],
        out_shape=[_f32((T, NC, n, F2)), _f32((n, 1)), _f32((n, 1))],
    )(degw, deg1, xs)

    Yall = spmv_w(src2, dst2, ew2,
                  xall.reshape(T * NC * n, F2)).reshape(T, NC, npad, F2)

    h, hs = pl.pallas_call(
        functools.partial(_t0_body, bn),
        grid=(nb,),
        in_specs=[spec_y(0), spec_t(0), specv(1), specv(1),
                  specw(F, 2 * F), specw(F, 2 * F), specw(F, F), specw(F, F),
                  specw(1, 2 * F), specw(1, F)],
        out_specs=[specv(), specs()],
        out_shape=[_f32((n, F)), _f32((NC, n, F2))],
    )(Yall, xs, dw, d1, Wi2, Wh2, W_hi, W_hh, b2, bh2)
    hsum = h

    for t in range(1, T):
        Q = spmv_1(src2, dst2, hs.reshape(NC * n, F2))
        z, rh, rhs = pl.pallas_call(
            _stepA_body,
            grid=(nb,),
            in_specs=[spec_y(t), spec_t(t), specv(1), specs(), specv(), specv(1),
                      specw(F, 2 * F), specw(F, 2 * F), specw(1, 2 * F)],
            out_specs=[specv(), specv(), specs()],
            out_shape=[_f32((n, F)), _f32((n, F)), _f32((NC, n, F2))],
        )(Yall, xs, dw, Q, h, d1, Wi2, Wh2, b2)
        S = spmv_1(src2, dst2, rhs.reshape(NC * n, F2))
        h, hs, hsum = pl.pallas_call(
            _stepB_body,
            grid=(nb,),
            in_specs=[spec_y(t), spec_t(t), specv(1), specs(), specv(), specv(),
                      specv(1), specw(F, F), specw(F, F), specw(1, F), specv()],
            out_specs=[specv(), specs(), specv()],
            out_shape=[_f32((n, F)), _f32((NC, n, F2)), _f32((n, F))],
        )(Yall, xs, dw, S, rh, z, d1, W_hi, W_hh, bh2, hsum)

    out = pl.pallas_call(
        functools.partial(_pool_body, G, C, bn),
        grid=(nb,),
        in_specs=[specv(), specv(1), specw(F, C), specw(1, C)],
        out_specs=pl.BlockSpec((G, C), lambda i: (0, 0)),
        out_shape=_f32((G, C)),
        scratch_shapes=[pltpu.VMEM((G, F), jnp.float32),
                        pltpu.VMEM((G, 1), jnp.float32)],
    )(hsum, batch2, lin_W, lin_b.reshape(1, C))
    return out


# speedup vs baseline: 1.0413x; 1.0413x over previous
import functools

import jax
import jax.numpy as jnp
from jax import lax
from jax.experimental import pallas as pl
from jax.experimental.pallas import tpu as pltpu
from jax.experimental.pallas import tpu_sc as plsc

NC = 2
NS = 16
K = 128
F = 128


def _mesh():
    return plsc.VectorSubcoreMesh(core_axis_name="c", subcore_axis_name="s")


def _make_deg(ch, npad):
    ntiles = NC * NS

    R = npad // 128

    def body(dst_hbm, w_hbm, outw_hbm, out1_hbm,
             dst_v, w_v, accw, acc1, acc2d, idx_v, shw, sh1):
        cid = lax.axis_index("c")
        sid = lax.axis_index("s")
        tile = cid * NS + sid
        zf = jnp.zeros((16,), jnp.float32)

        def zgrp(i, _):
            accw[pl.ds(i * 16, 16)] = zf
            acc1[pl.ds(i * 16, 16)] = zf
            return 0
        lax.fori_loop(0, npad // 16, zgrp, 0)
        for g in range(R // 16):
            idx_v[pl.ds(g * 16, 16)] = lax.iota(jnp.int32, 16) + g * 16
        @pl.when(sid < R // 8)
        def _():
            for c in range(8):
                for seg in range(128 // 16):
                    acc2d[c, pl.ds(seg * 16, 16)] = zf
            pltpu.sync_copy(acc2d.at[pl.ds(0, 8)], shw.at[pl.ds(sid * 8, 8)])
            pltpu.sync_copy(acc2d.at[pl.ds(0, 8)], sh1.at[pl.ds(sid * 8, 8)])
        pltpu.sync_copy(dst_hbm.at[pl.ds(tile * ch, ch)], dst_v)
        pltpu.sync_copy(w_hbm.at[pl.ds(tile * ch, ch)], w_v)
        ones16 = jnp.ones((16,), jnp.float32)

        def chunk(j, _):
            for e in range(K // 16):
                sl = pl.ds(e * 16, 16)
                d16 = dst_v[j, sl]
                plsc.addupdate_scatter(accw, [d16], w_v[j, sl])
                plsc.addupdate_scatter(acc1, [d16], ones16)
            return 0
        lax.fori_loop(0, ch, chunk, 0)
        plsc.subcore_barrier()

        def repack(acc_1d):
            def row(r, _):
                for seg in range(128 // 16):
                    acc2d[r, pl.ds(seg * 16, 16)] = \
                        acc_1d[pl.ds(r * 128 + seg * 16, 16)]
                return 0
            lax.fori_loop(0, R, row, 0)
        repack(accw)
        pltpu.sync_copy(acc2d, shw.at[idx_v], add=True)
        repack(acc1)
        pltpu.sync_copy(acc2d, sh1.at[idx_v], add=True)
        plsc.subcore_barrier()

        @pl.when(sid < R // 8)
        def _():
            pltpu.sync_copy(shw.at[pl.ds(sid * 8, 8)],
                            outw_hbm.at[cid, pl.ds(sid * 8, 8)])
            pltpu.sync_copy(sh1.at[pl.ds(sid * 8, 8)],
                            out1_hbm.at[cid, pl.ds(sid * 8, 8)])

    return pl.kernel(
        body,
        out_type=(jax.ShapeDtypeStruct((NC, R, 128), jnp.float32),
                  jax.ShapeDtypeStruct((NC, R, 128), jnp.float32)),
        mesh=_mesh(),
        compiler_params=pltpu.CompilerParams(needs_layout_passes=False),
        scratch_types=[
            pltpu.VMEM((ch, K), jnp.int32),
            pltpu.VMEM((ch, K), jnp.float32),
            pltpu.VMEM((npad,), jnp.float32),
            pltpu.VMEM((npad,), jnp.float32),
            pltpu.VMEM((R, 128), jnp.float32),
            pltpu.VMEM((R,), jnp.int32),
            pltpu.VMEM_SHARED((R, 128), jnp.float32),
            pltpu.VMEM_SHARED((R, 128), jnp.float32),
        ],
    )


F2 = F // 2


def _make_spmv(ch, n, npad, weighted, nt=1):
    rows_per_tile = npad // NS
    RING = 16

    def body(src_hbm, dst_hbm, *rest):
        if weighted:
            (w_hbm, x_hbm, y_hbm, srcr, dstr, wr,
             rows0, rows1, rows2, rows3,
             x_sh, accum, sem0, sem1, sem2) = rest
        else:
            (x_hbm, y_hbm, srcr, dstr,
             rows0, rows1, rows2, rows3,
             x_sh, accum, sem0, sem1, sem2) = rest
        cid = lax.axis_index("c")
        sid = lax.axis_index("s")
        zf = jnp.zeros((16,), jnp.float32)
        ebase = sid * ch
        nload = n // 10

        def stage(cbase, rbase, sync):
            srcd = pltpu.make_async_copy(
                src_hbm.at[pl.ds(ebase + cbase, 8)],
                srcr.at[pl.ds(rbase, 8)], sem2)
            srcd.start()
            dstd = pltpu.make_async_copy(
                dst_hbm.at[pl.ds(ebase + cbase, 8)],
                dstr.at[pl.ds(rbase, 8)], sem2)
            dstd.start()
            if weighted:
                wd = pltpu.make_async_copy(
                    w_hbm.at[pl.ds(ebase + cbase, 8)],
                    wr.at[pl.ds(rbase, 8)], sem2)
                wd.start()
            if sync:
                srcd.wait()
                dstd.wait()
                if weighted:
                    wd.wait()

        def wait_stage():
            pltpu.make_async_copy(src_hbm.at[pl.ds(0, 8)],
                                  srcr.at[pl.ds(0, 8)], sem2).wait()
            pltpu.make_async_copy(dst_hbm.at[pl.ds(0, 8)],
                                  dstr.at[pl.ds(0, 8)], sem2).wait()
            if weighted:
                pltpu.make_async_copy(w_hbm.at[pl.ds(0, 8)],
                                      wr.at[pl.ds(0, 8)], sem2).wait()

        def scale(rows_v, jrow):
            def grp(g, _):
                wv = wr[jrow, pl.ds(g * 16, 16)]
                for e16 in range(16):
                    ws = wv[e16]
                    row = g * 16 + e16
                    for c in range(F2 // 16):
                        sl = pl.ds(c * 16, 16)
                        rows_v[row, sl] = rows_v[row, sl] * ws
                return 0
            lax.fori_loop(0, K // 16, grp, 0)

        rows = (rows0, rows1, rows2, rows3)
        npair = ch // 8

        def halfgrp(slot4):
            for b in range(4):
                pltpu.make_async_copy(x_sh.at[srcr.at[slot4 + b]], rows[b],
                                      sem0).wait()
                if weighted:
                    scale(rows[b], slot4 + b)
                pltpu.async_copy(rows[b], accum.at[dstr.at[slot4 + b]], sem1,
                                 add=True)
            for b in range(4):
                pltpu.make_async_copy(rows[b], accum.at[dstr.at[slot4 + b]],
                                      sem1).wait()

        def firegrp(slot4):
            for b in range(4):
                pltpu.async_copy(x_sh.at[srcr.at[slot4 + b]], rows[b], sem0)

        def pairgrp(p, _):
            slot = lax.rem(p, 2) * 8
            oslot = 8 - slot

            @pl.when(p + 1 < npair)
            def _():
                wait_stage()

            halfgrp(slot)
            firegrp(slot + 4)
            halfgrp(slot + 4)

            @pl.when(p + 1 < npair)
            def _():
                firegrp(oslot)

            @pl.when(p + 2 < npair)
            def _():
                stage(p * 8 + 16, slot, False)
            return 0

        def phase(tt, _):
            def zrow(i, __):
                for c in range(F2 // 16):
                    rows0[i, pl.ds(c * 16, 16)] = zf
                return 0
            lax.fori_loop(0, K, zrow, 0)
            for b in range(rows_per_tile // K):
                pltpu.sync_copy(
                    rows0, accum.at[pl.ds(sid * rows_per_tile + b * K, K)])
            @pl.when(sid < 10)
            def _():
                pltpu.sync_copy(
                    x_hbm.at[pl.ds((tt * NC + cid) * n + sid * nload, nload)],
                    x_sh.at[pl.ds(sid * nload, nload)])
            stage(0, 0, True)
            stage(8, 8, False)
            plsc.subcore_barrier()
            firegrp(0)
            lax.fori_loop(0, npair, pairgrp, 0)
            plsc.subcore_barrier()
            pltpu.sync_copy(
                accum.at[pl.ds(sid * rows_per_tile, rows_per_tile)],
                y_hbm.at[tt * NC + cid,
                         pl.ds(sid * rows_per_tile, rows_per_tile)])
            return 0
        lax.fori_loop(0, nt, phase, 0)

    scratch = [
        pltpu.VMEM((RING, K), jnp.int32),
        pltpu.VMEM((RING, K), jnp.int32),
    ]
    if weighted:
        scratch.append(pltpu.VMEM((RING, K), jnp.float32))
    scratch += [
        pltpu.VMEM((K, F2), jnp.float32)] * 4 + [
        pltpu.VMEM_SHARED((n, F2), jnp.float32),
        pltpu.VMEM_SHARED((npad, F2), jnp.float32),
        pltpu.SemaphoreType.DMA,
        pltpu.SemaphoreType.DMA,
        pltpu.SemaphoreType.DMA,
    ]
    return pl.kernel(
        body,
        out_type=jax.ShapeDtypeStruct((nt * NC, npad, F2), jnp.float32),
        mesh=_mesh(),
        compiler_params=pltpu.CompilerParams(needs_layout_passes=False,
                                             use_tc_tiling_on_sc=False),
        scratch_types=scratch,
    )


def _prescale_body(degw_ref, deg1_ref, xs_ref,
                   x0_ref, x1_ref, x2_ref, x3_ref, dw_ref, d1_ref):
    dw = lax.rsqrt(degw_ref[0] + degw_ref[1] + 1.0)
    d1 = lax.rsqrt(deg1_ref[0] + deg1_ref[1] + 1.0)
    dw_ref[...] = dw
    d1_ref[...] = d1
    for t, r in enumerate((x0_ref, x1_ref, x2_ref, x3_ref)):
        xt = xs_ref[t] * dw
        r[0] = xt[:, :F2]
        r[1] = xt[:, F2:]


def _halves(y_ref, x, dinv):
    y = jnp.concatenate([y_ref[0], y_ref[1]], axis=1)
    return dinv * y + dinv * dinv * x


def _t0_body(bn, y_ref, x_ref, dw_ref, d1_ref,
             Wi2_ref, Wh2_ref, Whi_ref, Whh_ref, b2_ref, bh_ref,
             h_ref, hs_ref):
    i = pl.program_id(0)
    rows = lax.broadcasted_iota(jnp.int32, (bn, F), 0) + i * bn
    cols = lax.broadcasted_iota(jnp.int32, (bn, F), 1)
    eye = (rows == cols).astype(jnp.float32)
    P = _halves(y_ref, x_ref[0], dw_ref[...])
    zr = jax.nn.sigmoid(
        jnp.dot(P, Wi2_ref[...], preferred_element_type=jnp.float32)
        + jnp.dot(eye, Wh2_ref[...], preferred_element_type=jnp.float32)
        + b2_ref[...])
    z = zr[:, :F]
    r = zr[:, F:]
    rh = r * eye
    cand = jnp.tanh(
        jnp.dot(P, Whi_ref[...], preferred_element_type=jnp.float32)
        + jnp.dot(rh, Whh_ref[...], preferred_element_type=jnp.float32)
        + bh_ref[...])
    h = (1.0 - z) * rh + z * cand
    h_ref[...] = h
    hsc = d1_ref[...] * h
    hs_ref[0] = hsc[:, :F2]
    hs_ref[1] = hsc[:, F2:]


def _stepA_body(y_ref, x_ref, dw_ref, q_ref, h_ref, d1_ref,
                Wi2_ref, Wh2_ref, b2_ref,
                z_ref, rh_ref, rhs_ref):
    d1 = d1_ref[...]
    P = _halves(y_ref, x_ref[0], dw_ref[...])
    Q = _halves(q_ref, h_ref[...], d1)
    zr = jax.nn.sigmoid(
        jnp.dot(P, Wi2_ref[...], preferred_element_type=jnp.float32)
        + jnp.dot(Q, Wh2_ref[...], preferred_element_type=jnp.float32)
        + b2_ref[...])
    z = zr[:, :F]
    r = zr[:, F:]
    rh = r * h_ref[...]
    z_ref[...] = z
    rh_ref[...] = rh
    rhsc = d1 * rh
    rhs_ref[0] = rhsc[:, :F2]
    rhs_ref[1] = rhsc[:, F2:]


def _stepB_body(y_ref, x_ref, dw_ref, s_ref, rh_ref, z_ref, d1_ref,
                Whi_ref, Whh_ref, bh_ref, hsum_ref,
                h_ref, hs_ref, hsumo_ref):
    d1 = d1_ref[...]
    P = _halves(y_ref, x_ref[0], dw_ref[...])
    S = _halves(s_ref, rh_ref[...], d1)
    cand = jnp.tanh(
        jnp.dot(P, Whi_ref[...], preferred_element_type=jnp.float32)
        + jnp.dot(S, Whh_ref[...], preferred_element_type=jnp.float32)
        + bh_ref[...])
    z = z_ref[...]
    h = (1.0 - z) * rh_ref[...] + z * cand
    h_ref[...] = h
    hsc = d1 * h
    hs_ref[0] = hsc[:, :F2]
    hs_ref[1] = hsc[:, F2:]
    hsumo_ref[...] = hsum_ref[...] + h


def _pool_body(g, c, bn, hsum_ref, batch_ref, linW_ref, linb_ref,
               out_ref, sums_sc, cnt_sc):
    i = pl.program_id(0)
    oh = (batch_ref[...] == lax.broadcasted_iota(jnp.int32, (bn, g), 1)
          ).astype(jnp.float32)
    part = lax.dot_general(oh, hsum_ref[...], (((0,), (0,)), ((), ())),
                           preferred_element_type=jnp.float32)
    pcnt = lax.dot_general(oh, jnp.ones((bn, 1), jnp.float32),
                           (((0,), (0,)), ((), ())),
                           preferred_element_type=jnp.float32)

    @pl.when(i == 0)
    def _():
        sums_sc[...] = jnp.zeros_like(sums_sc)
        cnt_sc[...] = jnp.zeros_like(cnt_sc)
    sums_sc[...] += part
    cnt_sc[...] += pcnt

    @pl.when(i == pl.num_programs(0) - 1)
    def _():
        pooled = sums_sc[...] / jnp.maximum(cnt_sc[...], 1.0)
        out_ref[...] = (jnp.dot(pooled, linW_ref[...],
                                preferred_element_type=jnp.float32)
                        + linb_ref[...])


def _f32(shape):
    return jax.ShapeDtypeStruct(shape, jnp.float32)


def kernel(xs, edge_index, edge_weight, batch,
           W_zi, b_zi, W_zh, b_zh, W_ri, b_ri, W_rh, b_rh,
           W_hi, b_hi, W_hh, b_hh, lin_W, lin_b):
    T, n, f = xs.shape
    E = edge_index.shape[1]
    G = 32
    C = lin_W.shape[1]
    ntiles = NC * NS
    npad = -(-(n + 1) // (NS * K)) * (NS * K)
    ch = -(-E // (NS * K))
    ch = -(-ch // 16) * 16
    epad = NS * ch * K
    ch_deg = epad // (ntiles * K)

    src2 = jnp.pad(edge_index[0], (0, epad - E)).reshape(epad // K, K)
    dst2 = jnp.pad(edge_index[1], (0, epad - E),
                   constant_values=n).reshape(epad // K, K)
    ew2 = jnp.pad(edge_weight, (0, epad - E)).reshape(epad // K, K)

    Wi2 = jnp.concatenate([W_zi, W_ri], axis=1)
    Wh2 = jnp.concatenate([W_zh, W_rh], axis=1)
    b2 = jnp.concatenate([b_zi + b_zh, b_ri + b_rh]).reshape(1, 2 * F)
    bh2 = (b_hi + b_hh).reshape(1, F)
    batch2 = batch.reshape(n, 1)

    bn = 1000 if n % 1000 == 0 else (n // (n // 1000))
    nb = n // bn

    degw_h, deg1_h = _make_deg(ch_deg, npad)(dst2, ew2)
    degw = degw_h.reshape(NC, npad)[:, :n].reshape(NC, n, 1)
    deg1 = deg1_h.reshape(NC, npad)[:, :n].reshape(NC, n, 1)

    spmv_w = _make_spmv(ch, n, npad, True)
    spmv_1 = _make_spmv(ch, n, npad, False)

    def spec2(last=1):
        return pl.BlockSpec((NC, bn, last), lambda i: (0, i, 0))

    def specs():
        return pl.BlockSpec((NC, bn, F2), lambda i: (0, i, 0))


    def spec_t(t):
        return pl.BlockSpec((1, bn, F), lambda i, _t=t: (_t, i, 0))

    def specv(last=F):
        return pl.BlockSpec((bn, last), lambda i: (i, 0))

    def specw(r, c):
        return pl.BlockSpec((r, c), lambda i: (0, 0))

    x0s, x1s, x2s, x3s, dw, d1 = pl.pallas_call(
        _prescale_body,
        grid=(nb,),
        in_specs=[spec2(), spec2(), pl.BlockSpec((T, bn, F), lambda i: (0, i, 0))],
        out_specs=[specs(), specs(), specs(), specs(), specv(1), specv(1)],
        out_shape=[_f32((NC, n, F2))] * 4 + [_f32((n, 1))] * 2,
    )(degw, deg1, xs)

    Y = [spmv_w(src2, dst2, ew2, xts.reshape(NC * n, F2))
         for xts in (x0s, x1s, x2s, x3s)]

    h, hs = pl.pallas_call(
        functools.partial(_t0_body, bn),
        grid=(nb,),
        in_specs=[specs(), spec_t(0), specv(1), specv(1),
                  specw(F, 2 * F), specw(F, 2 * F), specw(F, F), specw(F, F),
                  specw(1, 2 * F), specw(1, F)],
        out_specs=[specv(), specs()],
        out_shape=[_f32((n, F)), _f32((NC, n, F2))],
    )(Y[0], xs, dw, d1, Wi2, Wh2, W_hi, W_hh, b2, bh2)
    hsum = h

    for t in range(1, T):
        Q = spmv_1(src2, dst2, hs.reshape(NC * n, F2))
        z, rh, rhs = pl.pallas_call(
            _stepA_body,
            grid=(nb,),
            in_specs=[specs(), spec_t(t), specv(1), specs(), specv(), specv(1),
                      specw(F, 2 * F), specw(F, 2 * F), specw(1, 2 * F)],
            out_specs=[specv(), specv(), specs()],
            out_shape=[_f32((n, F)), _f32((n, F)), _f32((NC, n, F2))],
        )(Y[t], xs, dw, Q, h, d1, Wi2, Wh2, b2)
        S = spmv_1(src2, dst2, rhs.reshape(NC * n, F2))
        h, hs, hsum = pl.pallas_call(
            _stepB_body,
            grid=(nb,),
            in_specs=[specs(), spec_t(t), specv(1), specs(), specv(), specv(),
                      specv(1), specw(F, F), specw(F, F), specw(1, F), specv()],
            out_specs=[specv(), specs(), specv()],
            out_shape=[_f32((n, F)), _f32((NC, n, F2)), _f32((n, F))],
        )(Y[t], xs, dw, S, rh, z, d1, W_hi, W_hh, bh2, hsum)

    out = pl.pallas_call(
        functools.partial(_pool_body, G, C, bn),
        grid=(nb,),
        in_specs=[specv(), specv(1), specw(F, C), specw(1, C)],
        out_specs=pl.BlockSpec((G, C), lambda i: (0, 0)),
        out_shape=_f32((G, C)),
        scratch_shapes=[pltpu.VMEM((G, F), jnp.float32),
                        pltpu.VMEM((G, 1), jnp.float32)],
    )(hsum, batch2, lin_W, lin_b.reshape(1, C))
    return out
